# baseline (speedup 1.0000x reference)
import jax
import jax.numpy as jnp
from jax.experimental import pallas as pl
from jax.experimental.pallas import tpu as pltpu

VMD_K = 8
HIDDEN = 128
INPUT_SIZE = 512


def _vmdk_kernel(in_ref, w_ref, b_ref, k0_ref, wp_ref, ow_ref, ob_ref, out_ref):
    hm = jax.lax.dot_general(
        w_ref[:], in_ref[:],
        dimension_numbers=(((2,), (1,)), ((0,), (0,))),
        preferred_element_type=jnp.float32,
    )
    h = jnp.maximum(hm + b_ref[:], 0.0)
    sel = (h - k0_ref[:]) * wp_ref[:]
    val = jnp.sum(sel * ow_ref[:], axis=(0, 1), keepdims=True) + ob_ref[:]
    out_ref[:] = 1.0 / (1.0 + jnp.exp(-val))


def kernel(input, vmd_w, vmd_b, K_param, W_param, out_w, out_b):
    D = VMD_K * HIDDEN
    k_rows = K_param.reshape(-1, HIDDEN)
    w_r = W_param.reshape(VMD_K, HIDDEN)
    ow_r = out_w.reshape(VMD_K, HIDDEN)
    ob_r = out_b.reshape(1, 1)

    out = pl.pallas_call(
        _vmdk_kernel,
        grid=(1,),
        in_specs=[
            pl.BlockSpec((VMD_K, INPUT_SIZE), lambda i: (0, 0)),
            pl.BlockSpec((VMD_K, HIDDEN, INPUT_SIZE), lambda i: (0, 0, 0)),
            pl.BlockSpec((VMD_K, HIDDEN), lambda i: (0, 0)),
            pl.BlockSpec((VMD_K, HIDDEN), lambda i: (0, 0)),
            pl.BlockSpec((VMD_K, HIDDEN), lambda i: (0, 0)),
            pl.BlockSpec((VMD_K, HIDDEN), lambda i: (0, 0)),
            pl.BlockSpec((1, 1), lambda i: (0, 0)),
        ],
        out_specs=pl.BlockSpec((1, 1), lambda i: (0, 0)),
        out_shape=jax.ShapeDtypeStruct((1, 1), jnp.float32),
    )(input, vmd_w, vmd_b, k_rows, w_r, ow_r, ob_r)
    return out.reshape(1)


# speedup vs baseline: 7.4383x; 7.4383x over previous
import jax
import jax.numpy as jnp
from jax.experimental import pallas as pl
from jax.experimental.pallas import tpu as pltpu

VMD_K = 8
HIDDEN = 128
INPUT_SIZE = 512


def _vmdk_kernel(in_ref, w_ref, b_ref, k0_ref, wp_ref, ow_ref, ob_ref, out_ref):
    hm = jax.lax.dot_general(
        w_ref[:], in_ref[:],
        dimension_numbers=(((2,), (1,)), ((0,), (0,))),
        preferred_element_type=jnp.float32,
    )
    h = jnp.maximum(hm + b_ref[:], 0.0)
    sel = (h - k0_ref[:]) * wp_ref[:]
    val = jnp.sum(sel * ow_ref[:], axis=(0, 1), keepdims=True) + ob_ref[:]
    out_ref[:] = 1.0 / (1.0 + jnp.exp(-val))


def kernel(input, vmd_w, vmd_b, K_param, W_param, out_w, out_b):
    D = VMD_K * HIDDEN
    k_rows = K_param[0].reshape(VMD_K, HIDDEN)
    w_r = W_param.reshape(VMD_K, HIDDEN)
    ow_r = out_w.reshape(VMD_K, HIDDEN)
    ob_r = out_b.reshape(1, 1)

    out = pl.pallas_call(
        _vmdk_kernel,
        grid=(1,),
        in_specs=[
            pl.BlockSpec((VMD_K, INPUT_SIZE), lambda i: (0, 0)),
            pl.BlockSpec((VMD_K, HIDDEN, INPUT_SIZE), lambda i: (0, 0, 0)),
            pl.BlockSpec((VMD_K, HIDDEN), lambda i: (0, 0)),
            pl.BlockSpec((VMD_K, HIDDEN), lambda i: (0, 0)),
            pl.BlockSpec((VMD_K, HIDDEN), lambda i: (0, 0)),
            pl.BlockSpec((VMD_K, HIDDEN), lambda i: (0, 0)),
            pl.BlockSpec((1, 1), lambda i: (0, 0)),
        ],
        out_specs=pl.BlockSpec((1, 1), lambda i: (0, 0)),
        out_shape=jax.ShapeDtypeStruct((1, 1), jnp.float32),
    )(input, vmd_w, vmd_b, k_rows, w_r, ow_r, ob_r)
    return out.reshape(1)


# speedup vs baseline: 10.0385x; 1.3496x over previous
import jax
import jax.numpy as jnp
from jax.experimental import pallas as pl
from jax.experimental.pallas import tpu as pltpu

VMD_K = 8
HIDDEN = 128
INPUT_SIZE = 512
D = VMD_K * HIDDEN


def _vmdk_kernel(in_ref, w_ref, b_ref, k0_ref, wp_ref, ow_ref, ob_ref, out_ref):
    hm = jax.lax.dot_general(
        w_ref[:], in_ref[:],
        dimension_numbers=(((2,), (1,)), ((0,), (0,))),
        preferred_element_type=jnp.float32,
    )
    h = jnp.maximum(hm + b_ref[:], 0.0)
    hf = h.reshape(1, D)
    sel = (hf - k0_ref[0:1, :]) * wp_ref[:]
    val = jnp.sum(sel * ow_ref[:], axis=(0, 1), keepdims=True) + ob_ref[:]
    out_ref[:] = 1.0 / (1.0 + jnp.exp(-val))


def kernel(input, vmd_w, vmd_b, K_param, W_param, out_w, out_b):
    out = pl.pallas_call(
        _vmdk_kernel,
        grid=(1,),
        in_specs=[
            pl.BlockSpec((VMD_K, INPUT_SIZE), lambda i: (0, 0)),
            pl.BlockSpec((VMD_K, HIDDEN, INPUT_SIZE), lambda i: (0, 0, 0)),
            pl.BlockSpec((VMD_K, HIDDEN), lambda i: (0, 0)),
            pl.BlockSpec((8, D), lambda i: (0, 0)),
            pl.BlockSpec((1, D), lambda i: (0, 0)),
            pl.BlockSpec((1, D), lambda i: (0, 0)),
            pl.BlockSpec((1, 1), lambda i: (0, 0)),
        ],
        out_specs=pl.BlockSpec((1, 1), lambda i: (0, 0)),
        out_shape=jax.ShapeDtypeStruct((1, 1), jnp.float32),
    )(
        input, vmd_w, vmd_b, K_param,
        W_param.reshape(1, D),
        out_w,
        out_b.reshape(1, 1),
    )
    return out.reshape(1)


# speedup vs baseline: 11.8480x; 1.1802x over previous
import jax
import jax.numpy as jnp
from jax.experimental import pallas as pl
from jax.experimental.pallas import tpu as pltpu

VMD_K = 8
HIDDEN = 128
INPUT_SIZE = 512
D = VMD_K * HIDDEN


def _vmdk_kernel(in_ref, w_ref, b_ref, k0_ref, wp_ref, ow_ref, ob_ref, out_ref):
    acc = jnp.zeros((1, HIDDEN), jnp.float32)
    for k in range(VMD_K):
        lo = k * HIDDEN
        hm = jax.lax.dot_general(
            in_ref[k:k + 1, :], w_ref[k],
            dimension_numbers=(((1,), (1,)), ((), ())),
            preferred_element_type=jnp.float32,
        )
        h = jnp.maximum(hm + b_ref[k:k + 1, :], 0.0)
        sel = (h - k0_ref[0:1, lo:lo + HIDDEN]) * wp_ref[:, lo:lo + HIDDEN]
        acc = acc + sel * ow_ref[:, lo:lo + HIDDEN]
    val = jnp.sum(acc, axis=(0, 1), keepdims=True) + ob_ref[:]
    out_ref[:] = 1.0 / (1.0 + jnp.exp(-val))


def kernel(input, vmd_w, vmd_b, K_param, W_param, out_w, out_b):
    out = pl.pallas_call(
        _vmdk_kernel,
        grid=(1,),
        in_specs=[
            pl.BlockSpec((VMD_K, INPUT_SIZE), lambda i: (0, 0)),
            pl.BlockSpec((VMD_K, HIDDEN, INPUT_SIZE), lambda i: (0, 0, 0)),
            pl.BlockSpec((VMD_K, HIDDEN), lambda i: (0, 0)),
            pl.BlockSpec((8, D), lambda i: (0, 0)),
            pl.BlockSpec((1, D), lambda i: (0, 0)),
            pl.BlockSpec((1, D), lambda i: (0, 0)),
            pl.BlockSpec((1, 1), lambda i: (0, 0)),
        ],
        out_specs=pl.BlockSpec((1, 1), lambda i: (0, 0)),
        out_shape=jax.ShapeDtypeStruct((1, 1), jnp.float32),
    )(
        input, vmd_w, vmd_b, K_param,
        W_param.reshape(1, D),
        out_w,
        out_b.reshape(1, 1),
    )
    return out.reshape(1)


# speedup vs baseline: 12.0080x; 1.0135x over previous
import jax
import jax.numpy as jnp
from jax.experimental import pallas as pl
from jax.experimental.pallas import tpu as pltpu

VMD_K = 8
HIDDEN = 128
INPUT_SIZE = 512
D = VMD_K * HIDDEN
STEPS = 2
K_PER_STEP = VMD_K // STEPS
D_PER_STEP = K_PER_STEP * HIDDEN


def _vmdk_kernel(in_ref, w_ref, b_ref, k0_ref, wp_ref, ow_ref, ob_ref,
                 out_ref, acc_ref):
    i = pl.program_id(0)
    acc = jnp.zeros((1, HIDDEN), jnp.float32)
    for k in range(K_PER_STEP):
        lo = k * HIDDEN
        hm = jax.lax.dot_general(
            in_ref[pl.ds(i * K_PER_STEP + k, 1), :], w_ref[k],
            dimension_numbers=(((1,), (1,)), ((), ())),
            preferred_element_type=jnp.float32,
        )
        h = jnp.maximum(hm + b_ref[pl.ds(i * K_PER_STEP + k, 1), :], 0.0)
        sel = (h - k0_ref[0:1, lo:lo + HIDDEN]) * wp_ref[:, lo:lo + HIDDEN]
        acc = acc + sel * ow_ref[:, lo:lo + HIDDEN]

    @pl.when(i == 0)
    def _():
        acc_ref[:] = acc

    @pl.when(i > 0)
    def _():
        val = jnp.sum(acc_ref[:] + acc, axis=(0, 1), keepdims=True) + ob_ref[:]
        out_ref[:] = 1.0 / (1.0 + jnp.exp(-val))


def kernel(input, vmd_w, vmd_b, K_param, W_param, out_w, out_b):
    out = pl.pallas_call(
        _vmdk_kernel,
        grid=(STEPS,),
        in_specs=[
            pl.BlockSpec((VMD_K, INPUT_SIZE), lambda i: (0, 0)),
            pl.BlockSpec((K_PER_STEP, HIDDEN, INPUT_SIZE), lambda i: (i, 0, 0)),
            pl.BlockSpec((VMD_K, HIDDEN), lambda i: (0, 0)),
            pl.BlockSpec((8, D_PER_STEP), lambda i: (0, i)),
            pl.BlockSpec((1, D_PER_STEP), lambda i: (0, i)),
            pl.BlockSpec((1, D_PER_STEP), lambda i: (0, i)),
            pl.BlockSpec((1, 1), lambda i: (0, 0)),
        ],
        out_specs=pl.BlockSpec((1, 1), lambda i: (0, 0)),
        out_shape=jax.ShapeDtypeStruct((1, 1), jnp.float32),
        scratch_shapes=[pltpu.VMEM((1, HIDDEN), jnp.float32)],
    )(
        input, vmd_w, vmd_b, K_param,
        W_param.reshape(1, D),
        out_w,
        out_b.reshape(1, 1),
    )
    return out.reshape(1)
